# Initial kernel scaffold; baseline (speedup 1.0000x reference)
import jax
import jax.numpy as jnp
from jax.experimental import pallas as pl


def kernel(coordinate, labels, center):
    raise NotImplementedError("write your pallas kernel here")


# speedup vs baseline: 3.9007x; 3.9007x over previous
import jax
import jax.numpy as jnp
from jax.experimental import pallas as pl
from jax.experimental.pallas import tpu as pltpu

_B = 16384
_FEAT = 128
_CPAD = 16
_BLK = 1024
_GRID = _B // _BLK


def _body(x_ref, lab_ref, c_ref, out_ref, acc_ref):
    i = pl.program_id(0)
    lab = lab_ref[...]
    classes = jax.lax.broadcasted_iota(jnp.int32, (_BLK, _CPAD), 1)
    onehot = (lab == classes).astype(jnp.float32)
    centers = jnp.dot(onehot, c_ref[...], preferred_element_type=jnp.float32)
    diff = x_ref[...] - centers
    d2 = jnp.sum(diff * diff, axis=1, keepdims=True)
    dist = jnp.sqrt(d2)
    part = jnp.concatenate(
        [
            jnp.sum(onehot * dist, axis=0, keepdims=True),
            jnp.sum(onehot, axis=0, keepdims=True),
        ],
        axis=0,
    )
    prev = jnp.where(i == 0, jnp.zeros_like(part), acc_ref[...])
    acc = prev + part
    acc_ref[...] = acc

    @pl.when(i == pl.num_programs(0) - 1)
    def _():
        s = acc[0:1, :]
        cnt = acc[1:2, :]
        contrib = jnp.where(cnt > 0.0, s / cnt, 0.0)
        out_ref[0, 0] = jnp.sum(contrib) / _B


def kernel(coordinate, labels, center):
    lab2d = labels.reshape(_B, 1)
    cpad = jnp.zeros((_CPAD, _FEAT), jnp.float32).at[:9].set(center)
    out = pl.pallas_call(
        _body,
        grid=(_GRID,),
        in_specs=[
            pl.BlockSpec((_BLK, _FEAT), lambda i: (i, 0)),
            pl.BlockSpec((_BLK, 1), lambda i: (i, 0)),
            pl.BlockSpec((_CPAD, _FEAT), lambda i: (0, 0)),
        ],
        out_specs=pl.BlockSpec(memory_space=pltpu.SMEM),
        out_shape=jax.ShapeDtypeStruct((1, 1), jnp.float32),
        scratch_shapes=[pltpu.VMEM((2, _CPAD), jnp.float32)],
    )(coordinate, lab2d, cpad)
    return out[0, 0]


# speedup vs baseline: 4.3201x; 1.1075x over previous
import jax
import jax.numpy as jnp
from jax.experimental import pallas as pl
from jax.experimental.pallas import tpu as pltpu

_B = 16384
_FEAT = 128
_CPAD = 16
_BLK = 1024
_GRID = _B // _BLK


def _body(x_ref, lab_ref, ct_ref, out_ref, acc_ref):
    i = pl.program_id(0)
    x = x_ref[...]
    ct = ct_ref[...]
    j = jax.lax.rem(i, 8)
    lab = lab_ref[0, pl.ds(j, 1), :]

    xc = jnp.dot(x, ct, preferred_element_type=jnp.float32)
    ones_f = jnp.ones((_FEAT, 1), jnp.float32)
    rown = jnp.dot(x * x, ones_f, preferred_element_type=jnp.float32)
    cn2 = jnp.sum(ct * ct, axis=0, keepdims=True)
    d2 = jnp.maximum(rown + cn2 - 2.0 * xc, 0.0)
    dist = jnp.sqrt(d2)

    classes = jax.lax.broadcasted_iota(jnp.int32, (_CPAD, _BLK), 0)
    onehot_t = (lab == classes).astype(jnp.float32)
    m = jnp.dot(onehot_t, dist, preferred_element_type=jnp.float32)
    eye = (
        jax.lax.broadcasted_iota(jnp.int32, (_CPAD, _CPAD), 0)
        == jax.lax.broadcasted_iota(jnp.int32, (_CPAD, _CPAD), 1)
    ).astype(jnp.float32)
    s = jnp.sum(m * eye, axis=1, keepdims=True)
    ones_b = jnp.ones((_BLK, 1), jnp.float32)
    cnt = jnp.dot(onehot_t, ones_b, preferred_element_type=jnp.float32)

    part = jnp.concatenate([s, cnt], axis=1)
    prev = jnp.where(i == 0, jnp.zeros_like(part), acc_ref[...])
    acc = prev + part
    acc_ref[...] = acc

    @pl.when(i == pl.num_programs(0) - 1)
    def _():
        s_t = acc[:, 0:1]
        c_t = acc[:, 1:2]
        contrib = jnp.where(c_t > 0.0, s_t / c_t, 0.0)
        out_ref[0, 0] = jnp.sum(contrib) / _B


def kernel(coordinate, labels, center):
    lab3 = labels.reshape(2, 8, _BLK)
    ct = jnp.zeros((_FEAT, _CPAD), jnp.float32).at[:, :9].set(center.T)
    out = pl.pallas_call(
        _body,
        grid=(_GRID,),
        in_specs=[
            pl.BlockSpec((_BLK, _FEAT), lambda i: (i, 0)),
            pl.BlockSpec((1, 8, _BLK), lambda i: (i // 8, 0, 0)),
            pl.BlockSpec((_FEAT, _CPAD), lambda i: (0, 0)),
        ],
        out_specs=pl.BlockSpec(memory_space=pltpu.SMEM),
        out_shape=jax.ShapeDtypeStruct((1, 1), jnp.float32),
        scratch_shapes=[pltpu.VMEM((_CPAD, 2), jnp.float32)],
    )(coordinate, lab3, ct)
    return out[0, 0]


# speedup vs baseline: 5.1801x; 1.1991x over previous
import jax
import jax.numpy as jnp
from jax.experimental import pallas as pl
from jax.experimental.pallas import tpu as pltpu

_B = 16384
_FEAT = 128
_CPAD = 16
_BLK = 1024
_GRID = _B // _BLK


def _body(x_ref, lab_ref, c_ref, out_ref, acc_ref):
    i = pl.program_id(0)
    x = x_ref[...]
    c = c_ref[...]
    cpad = jnp.concatenate([c, jnp.zeros((_CPAD - 9, _FEAT), jnp.float32)], axis=0)
    j = jax.lax.rem(i, 8)
    lab = lab_ref[0, pl.ds(j, 1), :]

    g = jax.lax.dot_general(
        x, cpad, (((1,), (1,)), ((), ())), preferred_element_type=jnp.float32
    )
    rown = jnp.dot(
        x * x, jnp.ones((_FEAT, 1), jnp.float32), preferred_element_type=jnp.float32
    )
    cn2 = jnp.sum(cpad * cpad, axis=1, keepdims=True)

    g_t = jax.lax.transpose(g, (1, 0))
    rown_t = jax.lax.transpose(rown, (1, 0))
    d2_t = jnp.maximum(rown_t + cn2 - 2.0 * g_t, 0.0)
    dist_t = jnp.sqrt(d2_t)

    classes = jax.lax.broadcasted_iota(jnp.int32, (_CPAD, _BLK), 0)
    onehot_t = (lab == classes).astype(jnp.float32)
    s = jnp.sum(onehot_t * dist_t, axis=1, keepdims=True)
    cnt = jnp.sum(onehot_t, axis=1, keepdims=True)

    part = jnp.concatenate([s, cnt], axis=1)
    prev = jnp.where(i == 0, jnp.zeros_like(part), acc_ref[...])
    acc = prev + part
    acc_ref[...] = acc

    @pl.when(i == pl.num_programs(0) - 1)
    def _():
        s_t = acc[:, 0:1]
        c_t = acc[:, 1:2]
        contrib = jnp.where(c_t > 0.0, s_t / c_t, 0.0)
        out_ref[0, 0] = jnp.sum(contrib) / _B


def kernel(coordinate, labels, center):
    lab3 = labels.reshape(2, 8, _BLK)
    out = pl.pallas_call(
        _body,
        grid=(_GRID,),
        in_specs=[
            pl.BlockSpec((_BLK, _FEAT), lambda i: (i, 0)),
            pl.BlockSpec((1, 8, _BLK), lambda i: (i // 8, 0, 0)),
            pl.BlockSpec((9, _FEAT), lambda i: (0, 0)),
        ],
        out_specs=pl.BlockSpec(memory_space=pltpu.SMEM),
        out_shape=jax.ShapeDtypeStruct((1, 1), jnp.float32),
        scratch_shapes=[pltpu.VMEM((_CPAD, 2), jnp.float32)],
    )(coordinate, lab3, center)
    return out[0, 0]


# speedup vs baseline: 9.9417x; 1.9192x over previous
import jax
import jax.numpy as jnp
from jax.experimental import pallas as pl
from jax.experimental.pallas import tpu as pltpu

_B = 16384
_FEAT = 128
_CPAD = 16
_BLK = 2048
_GRID = _B // _BLK

_DN_T = (((1,), (1,)), ((), ()))


def _body(x_ref, lab_ref, c_ref, out_ref, acc_ref):
    i = pl.program_id(0)
    x = x_ref[...]
    c = c_ref[...]
    cpad = jnp.concatenate([c, jnp.zeros((_CPAD - 9, _FEAT), jnp.float32)], axis=0)
    lab = lab_ref[0]

    g_t = jax.lax.dot_general(
        cpad, x, _DN_T, preferred_element_type=jnp.float32
    )
    rown_t = jax.lax.dot_general(
        jnp.ones((1, _FEAT), jnp.float32), x * x, _DN_T,
        preferred_element_type=jnp.float32,
    )
    cn2 = jnp.sum(cpad * cpad, axis=1, keepdims=True)

    d2_t = jnp.maximum(rown_t + cn2 - 2.0 * g_t, 0.0)
    dist_t = jnp.sqrt(d2_t)

    classes = jax.lax.broadcasted_iota(jnp.int32, (_CPAD, _BLK), 0)
    onehot_t = (lab == classes).astype(jnp.float32)
    s = jnp.sum(onehot_t * dist_t, axis=1, keepdims=True)
    cnt = jnp.sum(onehot_t, axis=1, keepdims=True)

    part = jnp.concatenate([s, cnt], axis=1)
    prev = jnp.where(i == 0, jnp.zeros_like(part), acc_ref[...])
    acc = prev + part
    acc_ref[...] = acc

    @pl.when(i == pl.num_programs(0) - 1)
    def _():
        s_t = acc[:, 0:1]
        c_t = acc[:, 1:2]
        contrib = jnp.where(c_t > 0.0, s_t / c_t, 0.0)
        out_ref[0, 0] = jnp.sum(contrib) / _B


def kernel(coordinate, labels, center):
    lab3 = labels.reshape(_GRID, 1, _BLK)
    out = pl.pallas_call(
        _body,
        grid=(_GRID,),
        in_specs=[
            pl.BlockSpec((_BLK, _FEAT), lambda i: (i, 0)),
            pl.BlockSpec((1, 1, _BLK), lambda i: (i, 0, 0)),
            pl.BlockSpec((9, _FEAT), lambda i: (0, 0)),
        ],
        out_specs=pl.BlockSpec(memory_space=pltpu.SMEM),
        out_shape=jax.ShapeDtypeStruct((1, 1), jnp.float32),
        scratch_shapes=[pltpu.VMEM((_CPAD, 2), jnp.float32)],
    )(coordinate, lab3, center)
    return out[0, 0]


# speedup vs baseline: 13.1905x; 1.3268x over previous
import jax
import jax.numpy as jnp
from jax.experimental import pallas as pl
from jax.experimental.pallas import tpu as pltpu

_B = 16384
_FEAT = 128
_CPAD = 16
_BLK = 4096
_GRID = _B // _BLK

_DN_T = (((1,), (1,)), ((), ()))


def _body(x_ref, lab_ref, c_ref, out_ref, acc_ref):
    i = pl.program_id(0)
    x = x_ref[...]
    c = c_ref[...]
    cpad = jnp.concatenate([c, jnp.zeros((_CPAD - 9, _FEAT), jnp.float32)], axis=0)
    lab = lab_ref[0]

    g_t = jax.lax.dot_general(
        cpad, x, _DN_T, preferred_element_type=jnp.float32
    )
    rown_t = jax.lax.dot_general(
        jnp.ones((1, _FEAT), jnp.float32), x * x, _DN_T,
        preferred_element_type=jnp.float32,
    )
    cn2 = jnp.sum(cpad * cpad, axis=1, keepdims=True)

    d2_t = jnp.maximum(rown_t + cn2 - 2.0 * g_t, 0.0)
    dist_t = jnp.sqrt(d2_t)

    classes = jax.lax.broadcasted_iota(jnp.int32, (_CPAD, _BLK), 0)
    onehot_t = (lab == classes).astype(jnp.float32)
    s = jnp.sum(onehot_t * dist_t, axis=1, keepdims=True)
    cnt = jnp.sum(onehot_t, axis=1, keepdims=True)

    part = jnp.concatenate([s, cnt], axis=1)
    prev = jnp.where(i == 0, jnp.zeros_like(part), acc_ref[...])
    acc = prev + part
    acc_ref[...] = acc

    @pl.when(i == pl.num_programs(0) - 1)
    def _():
        s_t = acc[:, 0:1]
        c_t = acc[:, 1:2]
        contrib = jnp.where(c_t > 0.0, s_t / c_t, 0.0)
        out_ref[0, 0] = jnp.sum(contrib) / _B


def kernel(coordinate, labels, center):
    lab3 = labels.reshape(_GRID, 1, _BLK)
    out = pl.pallas_call(
        _body,
        grid=(_GRID,),
        in_specs=[
            pl.BlockSpec((_BLK, _FEAT), lambda i: (i, 0)),
            pl.BlockSpec((1, 1, _BLK), lambda i: (i, 0, 0)),
            pl.BlockSpec((9, _FEAT), lambda i: (0, 0)),
        ],
        out_specs=pl.BlockSpec(memory_space=pltpu.SMEM),
        out_shape=jax.ShapeDtypeStruct((1, 1), jnp.float32),
        scratch_shapes=[pltpu.VMEM((_CPAD, 2), jnp.float32)],
    )(coordinate, lab3, center)
    return out[0, 0]


# speedup vs baseline: 14.6432x; 1.1101x over previous
import jax
import jax.numpy as jnp
from jax.experimental import pallas as pl
from jax.experimental.pallas import tpu as pltpu

_B = 16384
_FEAT = 128
_CPAD = 16
_BLK = 8192
_GRID = _B // _BLK

_DN_T = (((1,), (1,)), ((), ()))


def _body(x_ref, lab_ref, c_ref, out_ref, acc_ref):
    i = pl.program_id(0)
    x = x_ref[...]
    c = c_ref[...]
    cpad = jnp.concatenate([c, jnp.zeros((_CPAD - 9, _FEAT), jnp.float32)], axis=0)
    lab = lab_ref[0]

    g_t = jax.lax.dot_general(
        cpad, x, _DN_T, preferred_element_type=jnp.float32
    )
    rown_t = jax.lax.dot_general(
        jnp.ones((1, _FEAT), jnp.float32), x * x, _DN_T,
        preferred_element_type=jnp.float32,
    )
    cn2 = jnp.sum(cpad * cpad, axis=1, keepdims=True)

    d2_t = jnp.maximum(rown_t + cn2 - 2.0 * g_t, 0.0)
    dist_t = jnp.sqrt(d2_t)

    classes = jax.lax.broadcasted_iota(jnp.int32, (_CPAD, _BLK), 0)
    onehot_t = (lab == classes).astype(jnp.float32)
    s = jnp.sum(onehot_t * dist_t, axis=1, keepdims=True)
    cnt = jnp.sum(onehot_t, axis=1, keepdims=True)

    part = jnp.concatenate([s, cnt], axis=1)
    prev = jnp.where(i == 0, jnp.zeros_like(part), acc_ref[...])
    acc = prev + part
    acc_ref[...] = acc

    @pl.when(i == pl.num_programs(0) - 1)
    def _():
        s_t = acc[:, 0:1]
        c_t = acc[:, 1:2]
        contrib = jnp.where(c_t > 0.0, s_t / c_t, 0.0)
        out_ref[0, 0] = jnp.sum(contrib) / _B


def kernel(coordinate, labels, center):
    lab3 = labels.reshape(_GRID, 1, _BLK)
    out = pl.pallas_call(
        _body,
        grid=(_GRID,),
        in_specs=[
            pl.BlockSpec((_BLK, _FEAT), lambda i: (i, 0)),
            pl.BlockSpec((1, 1, _BLK), lambda i: (i, 0, 0)),
            pl.BlockSpec((9, _FEAT), lambda i: (0, 0)),
        ],
        out_specs=pl.BlockSpec(memory_space=pltpu.SMEM),
        out_shape=jax.ShapeDtypeStruct((1, 1), jnp.float32),
        scratch_shapes=[pltpu.VMEM((_CPAD, 2), jnp.float32)],
    )(coordinate, lab3, center)
    return out[0, 0]
